# baseline (speedup 1.0000x reference)
import functools

import jax
import jax.numpy as jnp
from jax.experimental import pallas as pl


_ROW_BLOCK = 1024


def _colmin(x_ref, y_ref, n):
    blk = min(_ROW_BLOCK, n)
    y = y_ref[0]

    def step(i, acc):
        xb = x_ref[0, pl.ds(i * blk, blk), :]
        t = jax.lax.dot_general(
            xb, y, (((1,), (1,)), ((), ())),
            preferred_element_type=jnp.float32,
        )
        return jnp.minimum(acc, jnp.min(t, axis=0, keepdims=True))

    acc0 = jnp.full((1, n), jnp.inf, dtype=jnp.float32)
    return jax.lax.fori_loop(0, n // blk, step, acc0)


def _loss_kernel(xa_ref, ya_ref, xb_ref, yb_ref, out_ref, *, n, k):
    m2 = _colmin(xa_ref, ya_ref, n)
    m1 = _colmin(xb_ref, yb_ref, n)
    d = jnp.sqrt(jnp.maximum(jnp.concatenate([m1, m2], axis=0), 0.0))
    loss_cd = jnp.sum(d) * jnp.float32(1.0 / n)

    bits = jax.lax.bitcast_convert_type(d, jnp.int32)

    def bs(_, lohi):
        lo, hi = lohi
        mid = lo + (hi - lo + 1) // 2
        cnt = jnp.sum((bits >= mid).astype(jnp.int32), axis=1, keepdims=True)
        take = cnt >= k
        return (jnp.where(take, mid, lo), jnp.where(take, hi, mid - 1))

    lo0 = jnp.zeros((2, 1), jnp.int32)
    hi0 = jnp.full((2, 1), 0x7F000000, jnp.int32)
    lo, _ = jax.lax.fori_loop(0, 32, bs, (lo0, hi0))
    v = jax.lax.bitcast_convert_type(lo, jnp.float32)
    sum_gt = jnp.sum(jnp.where(d > v, d, 0.0))
    cnt_gt = jnp.sum((d > v).astype(jnp.float32), axis=1, keepdims=True)
    corr = jnp.sum((jnp.float32(k) - cnt_gt) * v)
    loss_w = (sum_gt + corr) * jnp.float32(1.0 / k)
    out_ref[0, 0, :] = jnp.full((128,), loss_cd + 3.0 * loss_w, jnp.float32)


def _hi_lo(x2):
    hi_f = jax.lax.bitcast_convert_type(
        jax.lax.bitcast_convert_type(x2, jnp.int32) & jnp.int32(-65536),
        jnp.float32)
    return hi_f.astype(jnp.bfloat16), (x2 - hi_f).astype(jnp.bfloat16)


def _operands(x, y):
    b, n, _ = x.shape
    x2 = jnp.sum(x * x, axis=-1, keepdims=True)
    y2 = jnp.sum(y * y, axis=-1, keepdims=True)
    x2hi, x2lo = _hi_lo(x2)
    y2hi, y2lo = _hi_lo(y2)
    ones = jnp.ones((b, n, 1), jnp.bfloat16)
    zpad = jnp.zeros((b, n, 121), jnp.bfloat16)
    xa = jnp.concatenate(
        [-2.0 * x.astype(jnp.bfloat16), x2hi, x2lo, ones, ones, zpad],
        axis=-1)
    ya = jnp.concatenate(
        [y.astype(jnp.bfloat16), ones, ones, y2hi, y2lo, zpad],
        axis=-1)
    return xa, ya


def kernel(pred_pointclouds, gt_pointclouds):
    pred = pred_pointclouds.astype(jnp.float32)
    gt = gt_pointclouds.astype(jnp.float32)
    b, n, _ = pred.shape
    k = int(0.5 * n)

    xa, ya = _operands(pred, gt)
    xb, yb = _operands(gt, pred)

    spec = pl.BlockSpec((1, n, 128), lambda i: (i, 0, 0))
    losses = pl.pallas_call(
        functools.partial(_loss_kernel, n=n, k=k),
        grid=(b,),
        in_specs=[spec, spec, spec, spec],
        out_specs=pl.BlockSpec((1, 1, 128), lambda i: (i, 0, 0)),
        out_shape=jax.ShapeDtypeStruct((b, 1, 128), jnp.float32),
    )(xa, ya, xb, yb)
    return jnp.sum(losses[:, 0, 0]) / b
